# baseline (speedup 1.0000x reference)
import functools

import jax
import jax.numpy as jnp
from jax import lax
from jax.experimental import pallas as pl
from jax.experimental.pallas import tpu as pltpu
from jax.experimental.pallas import tpu_sc as plsc

N = 50000
NP = 50176
E = 800000
EP = 819200
D = 128
NR = 8
R = NP // NR
NC = 2
NS = 16
NW = NC * NS
RW = EP // (NW * 128)
TR = (EP // 128) // NS
WT = R // NS
ZR = 56
NB = 4

f32 = jnp.float32
i32 = jnp.int32


def _dot(a, b):
    return jnp.dot(a, b, preferred_element_type=f32)


def _gvp(s, v, Wh, Ws, bs, Wv):
    vh = [_dot(vc, Wh) for vc in v]
    vn = jnp.sqrt(jnp.maximum(vh[0] * vh[0] + vh[1] * vh[1] + vh[2] * vh[2], 1e-8))
    so = _dot(jnp.concatenate([s, vn], axis=1), Ws) + bs
    vo = None if Wv is None else [_dot(h, Wv) for h in vh]
    return so, vo


def _gvp1(s, v, wh, Ws, bs, wv):
    vh = [vc * wh for vc in v]
    vn = jnp.sqrt(jnp.maximum(vh[0] * vh[0] + vh[1] * vh[1] + vh[2] * vh[2], 1e-8))
    so = _dot(jnp.concatenate([s, vn], axis=1), Ws) + bs
    return so, [h * wv for h in vh]


def _ln(s, v, g, b):
    mu = jnp.mean(s, axis=1, keepdims=True)
    var = jnp.mean((s - mu) * (s - mu), axis=1, keepdims=True)
    so = (s - mu) / jnp.sqrt(var + 1e-5) * g + b
    nk = jnp.maximum(v[0] * v[0] + v[1] * v[1] + v[2] * v[2], 1e-8)
    rms = jnp.sqrt(jnp.mean(nk, axis=1, keepdims=True))
    return so, [vc / rms for vc in v]


def _k1_body(xs, xv, nt, wh_np, ws_np, bs_np, wv_np, g_npln, b_npln, emb_n,
             g_gn, b_gn, wh_gn, ws_gn, bs_gn, wv_gn,
             wha, whc, wsa, wsc, t_out, ts_out, td_out):
    v = [xv[:, 3 * c:3 * c + 3] for c in range(3)]
    s, v = _gvp(xs[...], v, wh_np[...], ws_np[...], bs_np[...], wv_np[...])
    s, v = _ln(s, v, g_npln[...], b_npln[...])
    oh = (nt[...] == lax.broadcasted_iota(i32, (nt.shape[0], 20), 1)).astype(f32)
    s = jnp.concatenate([_dot(oh, emb_n[...]), s], axis=1)
    s, v = _ln(s, v, g_gn[...], b_gn[...])
    s, v = _gvp(s, v, wh_gn[...], ws_gn[...], bs_gn[...], wv_gn[...])
    z = jnp.zeros((s.shape[0], 13), f32)
    t_out[...] = jnp.concatenate(
        [s, v[0], v[1], v[2], jnp.zeros((s.shape[0], 40), f32)], axis=1)
    ts_out[...] = jnp.concatenate(
        [_dot(s, wsa[...])] + [_dot(vc, wha[...]) for vc in v] + [z], axis=1)
    td_out[...] = jnp.concatenate(
        [_dot(s, wsc[...])] + [_dot(vc, whc[...]) for vc in v] + [z], axis=1)


def _node_pre(xs, xv, nt, w):
    B = 1024
    grid = NP // B
    row = lambda width: pl.BlockSpec((B, width), lambda i: (i, 0))
    full = lambda a: pl.BlockSpec(a.shape, lambda i: (0,) * a.ndim)
    data = [xs, xv, nt] + w
    return pl.pallas_call(
        _k1_body,
        grid=(grid,),
        in_specs=[row(6), row(9), row(1)] + [full(a) for a in w],
        out_specs=[row(D)] * 3,
        out_shape=[jax.ShapeDtypeStruct((NP, D), f32)] * 3,
    )(*data)


def _k2_body(ts_hbm, td_hbm, src_hbm, dst_hbm, gs_hbm, gd_hbm, idx_v, rows_v,
             *sems):
    gsem = sems[:4]
    ssem = sems[4:]
    wid = lax.axis_index("s") * NC + lax.axis_index("c")
    for t_hbm, idx_hbm, out_hbm in ((ts_hbm, src_hbm, gs_hbm),
                                    (td_hbm, dst_hbm, gd_hbm)):
        pltpu.sync_copy(idx_hbm.at[pl.ds(wid * RW, RW)], idx_v)

        def fire(j, b):
            pltpu.async_copy(t_hbm.at[idx_v.at[j]], rows_v.at[b], gsem[b])

        def store(b, j):
            pltpu.async_copy(rows_v.at[b],
                             out_hbm.at[pl.ds((wid * RW + j) * 128, 128)],
                             ssem[b])

        def wait_g(b):
            pltpu.make_async_copy(t_hbm.at[idx_v.at[0]], rows_v.at[b],
                                  gsem[b]).wait()

        def wait_s(b):
            pltpu.make_async_copy(rows_v.at[b],
                                  out_hbm.at[pl.ds(0, 128)], ssem[b]).wait()

        for b in range(2):
            fire(b, b)

        def body(jg, carry):
            for b in range(4):
                j = jg * 4 + b
                wait_g(b)
                store(b, j)
                jn = j + 2
                bn = (b + 2) % 4

                @pl.when(jn < RW)
                def _():
                    @pl.when(j >= 2)
                    def _():
                        wait_s(bn)
                    fire(jn, bn)
            return carry

        lax.fori_loop(0, RW // 4, body, 0)
        for b in range(4):
            wait_s(b)


def _edge_gather(ts, td, src2, dst2):
    mesh = plsc.VectorSubcoreMesh(core_axis_name="c", subcore_axis_name="s")
    fn = pl.kernel(
        _k2_body,
        out_type=(jax.ShapeDtypeStruct((EP, D), f32),
                  jax.ShapeDtypeStruct((EP, D), f32)),
        mesh=mesh,
        scratch_types=[pltpu.VMEM((RW, 128), i32),
                       pltpu.VMEM((4, 128, D), f32)]
                      + [pltpu.SemaphoreType.DMA] * 8,
    )
    return fn(ts, td, src2, dst2)


def _k3_body(gs, gd, ea_in, ev_in, et,
             wh_ep, ws_ep, bs_ep, wv_ep, g_epln, b_epln, emb_e,
             g_ge, b_ge, wh_ge, ws_ge, bs_ge, wv_ge,
             whrow, wb_m0, wd_m0, bs_m0, wv_m0,
             wh_m1, ws_m1, bs_m1, wv_m1,
             wh_m2, ws_m2, bs_m2, wv_m2,
             pay_out):
    n = gs.shape[0]
    ea = ea_in[...]
    ev = [ev_in[:, c:c + 1] for c in range(3)]
    es, ev = _gvp1(ea, ev, wh_ep[0, 0], ws_ep[...], bs_ep[...], wv_ep[0, 0])
    es, ev = _ln(es, ev, g_epln[...], b_epln[...])
    oh = (et[...] == lax.broadcasted_iota(i32, (n, 4), 1)).astype(f32)
    es = jnp.concatenate([_dot(oh, emb_e[...]), es], axis=1)
    es, ev = _ln(es, ev, g_ge[...], b_ge[...])
    es, ev = _gvp1(es, ev, wh_ge[0, 0], ws_ge[...], bs_ge[...], wv_ge[0, 0])

    gsr, gdr = gs[...], gd[...]
    vh = [gsr[:, 64 + 17 * c:81 + 17 * c] + ev[c] * whrow[...]
          + gdr[:, 64 + 17 * c:81 + 17 * c] for c in range(3)]
    vn = jnp.sqrt(jnp.maximum(
        vh[0] * vh[0] + vh[1] * vh[1] + vh[2] * vh[2], 1e-8))
    ms = (gsr[:, 0:64] + gdr[:, 0:64] + _dot(es, wb_m0[...])
          + _dot(vn, wd_m0[...]) + bs_m0[...])
    mv = [_dot(h, wv_m0[...]) for h in vh]
    ms, mv = _gvp(ms, mv, wh_m1[...], ws_m1[...], bs_m1[...], wv_m1[...])
    ms, mv = _gvp(ms, mv, wh_m2[...], ws_m2[...], bs_m2[...], wv_m2[...])

    pay_out[...] = jnp.concatenate(
        [ms, mv[0], mv[1], mv[2], jnp.ones((n, 1), f32),
         jnp.zeros((n, 39), f32)], axis=1)


def _messages(gs, gd, ea, ev, et, w):
    B = 1280
    grid = E // B
    row = lambda width: pl.BlockSpec((B, width), lambda i: (i, 0))
    full = lambda a: pl.BlockSpec(a.shape, lambda i: (0,) * a.ndim)
    return pl.pallas_call(
        _k3_body,
        grid=(grid,),
        in_specs=[row(D), row(D), row(32), row(3), row(1)]
                 + [full(a) for a in w],
        out_specs=row(D),
        out_shape=jax.ShapeDtypeStruct((EP, D), f32),
    )(gs, gd, ea, ev, et, *w)


def _k4_body(pay_hbm, dst_hbm, agg_hbm, dst_v, pay_v, idx_v, zero_v, acc,
             *sems):
    lsem = sems[:4]
    csem = sems[4:]
    c = lax.axis_index("c")
    t = lax.axis_index("s")

    def zb(i, carry):
        for k in range(8):
            zero_v[i, 16 * k:16 * k + 16] = jnp.zeros((16,), f32)
        return carry

    lax.fori_loop(0, ZR, zb, 0)

    def fire(j, b):
        e0 = (t * TR + j) * 128
        pltpu.async_copy(pay_hbm.at[pl.ds(e0, 128)], pay_v.at[b], lsem[b])
        pltpu.async_copy(dst_hbm.at[pl.ds(e0, 128)], dst_v.at[b], lsem[b])

    def wait_l(b):
        pltpu.make_async_copy(pay_hbm.at[pl.ds(0, 128)], pay_v.at[b],
                              lsem[b]).wait()
        pltpu.make_async_copy(dst_hbm.at[pl.ds(0, 128)], dst_v.at[b],
                              lsem[b]).wait()

    def wait_c(b):
        pltpu.make_async_copy(pay_v.at[b], acc.at[idx_v.at[b]],
                              csem[b]).wait()

    for r in range(NR):

        @pl.when((r // 4) == c)
        def _(r=r):
            base = r * R
            for z in range(WT // ZR):
                pltpu.sync_copy(zero_v, acc.at[pl.ds(t * WT + z * ZR, ZR)])
            plsc.subcore_barrier()
            for b in range(2):
                fire(b, b)

            def body(jg, carry):
                for b in range(4):
                    j = jg * 4 + b
                    wait_l(b)
                    for k in range(8):
                        loc = dst_v[b, 16 * k:16 * k + 16] - base
                        loc = jnp.where((loc < 0) | (loc >= R), R, loc)
                        idx_v[b, 16 * k:16 * k + 16] = loc
                    pltpu.async_copy(pay_v.at[b], acc.at[idx_v.at[b]],
                                     csem[b], add=True)
                    jn = j + 2
                    bn = (b + 2) % 4

                    @pl.when(jn < TR)
                    def _():
                        @pl.when(j >= 2)
                        def _():
                            wait_c(bn)
                        fire(jn, bn)
                return carry

            lax.fori_loop(0, TR // 4, body, 0)
            for b in range(4):
                wait_c(b)
            plsc.subcore_barrier()
            pltpu.sync_copy(acc.at[pl.ds(t * WT, WT)],
                            agg_hbm.at[pl.ds(base + t * WT, WT)])
            plsc.subcore_barrier()


def _scatter(pay, dst1):
    mesh = plsc.VectorSubcoreMesh(core_axis_name="c", subcore_axis_name="s")
    fn = pl.kernel(
        _k4_body,
        out_type=jax.ShapeDtypeStruct((NP, D), f32),
        mesh=mesh,
        scratch_types=[pltpu.VMEM((4, 128), i32),
                       pltpu.VMEM((4, 128, D), f32),
                       pltpu.VMEM((4, 128), i32),
                       pltpu.VMEM((ZR, D), f32),
                       pltpu.VMEM_SHARED((R + 8, D), f32)]
                      + [pltpu.SemaphoreType.DMA] * 8,
    )
    return fn(pay, dst1)


def _k5_body(t_in, agg_in,
             g_n0, b_n0, wh_f0, ws_f0, bs_f0, wv_f0,
             wh_f1, ws_f1, bs_f1, wv_f1,
             g_n1, b_n1, g_fl, b_fl, wh_o, ws_o, bs_o, out):
    agg = agg_in[...]
    tr = t_in[...]
    s = tr[:, 0:64]
    v = [tr[:, 64 + 8 * c:72 + 8 * c] for c in range(3)]
    cnt = jnp.maximum(agg[:, 88:89], 1.0)
    s = s + agg[:, 0:64] / cnt
    v = [v[c] + agg[:, 64 + 8 * c:72 + 8 * c] / cnt for c in range(3)]
    s, v = _ln(s, v, g_n0[...], b_n0[...])
    fs, fv = _gvp(s, v, wh_f0[...], ws_f0[...], bs_f0[...], wv_f0[...])
    fs, fv = _gvp(fs, fv, wh_f1[...], ws_f1[...], bs_f1[...], wv_f1[...])
    s, v = _ln(s + fs, [v[c] + fv[c] for c in range(3)], g_n1[...], b_n1[...])
    s, v = _ln(s, v, g_fl[...], b_fl[...])
    o, _ = _gvp(s, v, wh_o[...], ws_o[...], bs_o[...], None)
    out[...] = o


def _node_post(t, agg, w):
    B = 1024
    grid = NP // B
    row = lambda width: pl.BlockSpec((B, width), lambda i: (i, 0))
    full = lambda a: pl.BlockSpec(a.shape, lambda i: (0,) * a.ndim)
    return pl.pallas_call(
        _k5_body,
        grid=(grid,),
        in_specs=[row(D), row(D)] + [full(a) for a in w],
        out_specs=row(8),
        out_shape=jax.ShapeDtypeStruct((NP, 8), f32),
    )(t, agg, *w)


def _pad_rows(a, rows):
    return jnp.pad(a, ((0, rows - a.shape[0]),) + ((0, 0),) * (a.ndim - 1))


def kernel(x_s, x_v, edge_index, ntypes, etypes, eattr_s, eattr_v, params):
    p = params
    r2 = lambda a: a.reshape(1, -1)

    xs = _pad_rows(x_s.astype(f32), NP)
    xv = _pad_rows(x_v.astype(f32).transpose(0, 2, 1).reshape(N, 9), NP)
    nt = _pad_rows(ntypes.astype(i32).reshape(N, 1), NP)
    src2 = jnp.pad(edge_index[0].astype(i32), (0, EP - E),
                   constant_values=N).reshape(EP // 128, 128)
    dst1 = jnp.pad(edge_index[1].astype(i32), (0, EP - E), constant_values=N)
    dst2 = dst1.reshape(EP // 128, 128)
    et = etypes.astype(i32).reshape(E, 1)
    ea = eattr_s.astype(f32)
    ev = eattr_v.astype(f32).reshape(E, 3)

    lp = p['convs'][0]
    wh0, ws0 = lp['msg0']['Wh'], lp['msg0']['Ws']
    w1 = [p['np_gvp']['Wh'], p['np_gvp']['Ws'], r2(p['np_gvp']['bs']),
          p['np_gvp']['Wv'], r2(p['np_ln']['g']), r2(p['np_ln']['b']),
          p['ntype_emb'], r2(p['gn_ln']['g']), r2(p['gn_ln']['b']),
          p['gn_gvp']['Wh'], p['gn_gvp']['Ws'], r2(p['gn_gvp']['bs']),
          p['gn_gvp']['Wv'],
          wh0[0:8], wh0[9:17], ws0[0:64], ws0[96:160]]
    t, ts, td = _node_pre(xs, xv, nt, w1)

    gs, gd = _edge_gather(ts, td, src2, dst2)

    w3 = [p['ep_gvp']['Wh'], p['ep_gvp']['Ws'], r2(p['ep_gvp']['bs']),
          p['ep_gvp']['Wv'], r2(p['ep_ln']['g']), r2(p['ep_ln']['b']),
          p['etype_emb'], r2(p['ge_ln']['g']), r2(p['ge_ln']['b']),
          p['ge_gvp']['Wh'], p['ge_gvp']['Ws'], r2(p['ge_gvp']['bs']),
          p['ge_gvp']['Wv'],
          wh0[8:9], ws0[64:96], ws0[160:177], r2(lp['msg0']['bs']),
          lp['msg0']['Wv'],
          lp['msg1']['Wh'], lp['msg1']['Ws'], r2(lp['msg1']['bs']),
          lp['msg1']['Wv'],
          lp['msg2']['Wh'], lp['msg2']['Ws'], r2(lp['msg2']['bs']),
          lp['msg2']['Wv']]
    pay = _messages(gs, gd, ea, ev, et, w3)

    agg = _scatter(pay, dst1)

    w5 = [r2(lp['norm0']['g']), r2(lp['norm0']['b']),
          lp['ff0']['Wh'], lp['ff0']['Ws'], r2(lp['ff0']['bs']),
          lp['ff0']['Wv'],
          lp['ff1']['Wh'], lp['ff1']['Ws'], r2(lp['ff1']['bs']),
          lp['ff1']['Wv'],
          r2(lp['norm1']['g']), r2(lp['norm1']['b']),
          r2(p['final_ln']['g']), r2(p['final_ln']['b']),
          p['out_gvp']['Wh'], p['out_gvp']['Ws'], r2(p['out_gvp']['bs'])]
    out = _node_post(t, agg, w5)
    return out[:N]


# speedup vs baseline: 1.0406x; 1.0406x over previous
import functools

import jax
import jax.numpy as jnp
from jax import lax
from jax.experimental import pallas as pl
from jax.experimental.pallas import tpu as pltpu
from jax.experimental.pallas import tpu_sc as plsc

N = 50000
NP = 50176
E = 800000
EP = 819200
D = 128
NR = 8
R = NP // NR
NC = 2
NS = 16
NW = NC * NS
RW = EP // (NW * 128)
TR = (EP // 128) // NS
WT = R // NS
ZR = 56
NB = 4

f32 = jnp.float32
i32 = jnp.int32


def _dot(a, b):
    return jnp.dot(a, b, preferred_element_type=f32)


def _norms(vhs, H):
    return (vhs[:, 0:H] * vhs[:, 0:H] + vhs[:, H:2 * H] * vhs[:, H:2 * H]
            + vhs[:, 2 * H:3 * H] * vhs[:, 2 * H:3 * H])


def _gvp_v(s, vs, H, whb, ws, bs, wvb):
    vhs = _dot(vs, whb)
    vn = jnp.sqrt(jnp.maximum(_norms(vhs, H), 1e-8))
    so = _dot(jnp.concatenate([s, vn], axis=1), ws) + bs
    vo = None if wvb is None else _dot(vhs, wvb)
    return so, vo


def _gvp1(s, vs, wh, ws, bs, wv):
    vhs = vs * wh
    vn = jnp.sqrt(jnp.maximum(_norms(vhs, 1), 1e-8))
    so = _dot(jnp.concatenate([s, vn], axis=1), ws) + bs
    return so, vhs * wv


def _ln_v(s, vs, K, g, b):
    mu = jnp.mean(s, axis=1, keepdims=True)
    var = jnp.mean((s - mu) * (s - mu), axis=1, keepdims=True)
    so = (s - mu) / jnp.sqrt(var + 1e-5) * g + b
    nk = jnp.maximum(_norms(vs, K), 1e-8)
    rms = jnp.sqrt(jnp.mean(nk, axis=1, keepdims=True))
    return so, vs / rms


def _k1_body(nin, wh_np, ws_np, bs_np, wv_np, g_npln, b_npln, emb_n,
             g_gn, b_gn, wh_gn, ws_gn, bs_gn, wv_gn,
             wha, whc, wsa, wsc, t_out, ts_out, td_out):
    r = nin[...]
    n = r.shape[0]
    xs, vs, ntf = r[:, 0:6], r[:, 6:15], r[:, 15:16]
    s, vs = _gvp_v(xs, vs, 8, wh_np[...], ws_np[...], bs_np[...], wv_np[...])
    s, vs = _ln_v(s, vs, 8, g_npln[...], b_npln[...])
    oh = (ntf.astype(i32) == lax.broadcasted_iota(i32, (n, 20), 1)).astype(f32)
    s = jnp.concatenate([_dot(oh, emb_n[...]), s], axis=1)
    s, vs = _ln_v(s, vs, 8, g_gn[...], b_gn[...])
    s, vs = _gvp_v(s, vs, 8, wh_gn[...], ws_gn[...], bs_gn[...], wv_gn[...])
    z = jnp.zeros((n, 13), f32)
    t_out[...] = jnp.concatenate([s, vs, jnp.zeros((n, 40), f32)], axis=1)
    ts_out[...] = jnp.concatenate(
        [_dot(s, wsa[...]), _dot(vs, wha[...]), z], axis=1)
    td_out[...] = jnp.concatenate(
        [_dot(s, wsc[...]), _dot(vs, whc[...]), z], axis=1)


def _node_pre(nin, w):
    B = 1024
    grid = NP // B
    row = lambda width: pl.BlockSpec((B, width), lambda i: (i, 0))
    full = lambda a: pl.BlockSpec(a.shape, lambda i: (0,) * a.ndim)
    return pl.pallas_call(
        _k1_body,
        grid=(grid,),
        in_specs=[row(16)] + [full(a) for a in w],
        out_specs=[row(D)] * 3,
        out_shape=[jax.ShapeDtypeStruct((NP, D), f32)] * 3,
    )(nin, *w)


def _k2_body(ts_hbm, td_hbm, src_hbm, dst_hbm, gs_hbm, gd_hbm, idx_v, rows_v,
             *sems):
    gsem = sems[:4]
    ssem = sems[4:]
    wid = lax.axis_index("s") * NC + lax.axis_index("c")
    for t_hbm, idx_hbm, out_hbm in ((ts_hbm, src_hbm, gs_hbm),
                                    (td_hbm, dst_hbm, gd_hbm)):
        pltpu.sync_copy(idx_hbm.at[pl.ds(wid * RW, RW)], idx_v)

        def fire(j, b):
            pltpu.async_copy(t_hbm.at[idx_v.at[j]], rows_v.at[b], gsem[b])

        def store(b, j):
            pltpu.async_copy(rows_v.at[b],
                             out_hbm.at[pl.ds((wid * RW + j) * 128, 128)],
                             ssem[b])

        def wait_g(b):
            pltpu.make_async_copy(t_hbm.at[idx_v.at[0]], rows_v.at[b],
                                  gsem[b]).wait()

        def wait_s(b):
            pltpu.make_async_copy(rows_v.at[b],
                                  out_hbm.at[pl.ds(0, 128)], ssem[b]).wait()

        for b in range(2):
            fire(b, b)

        def body(jg, carry):
            for b in range(4):
                j = jg * 4 + b
                wait_g(b)
                store(b, j)
                jn = j + 2
                bn = (b + 2) % 4

                @pl.when(jn < RW)
                def _():
                    @pl.when(j >= 2)
                    def _():
                        wait_s(bn)
                    fire(jn, bn)
            return carry

        lax.fori_loop(0, RW // 4, body, 0)
        for b in range(4):
            wait_s(b)


def _edge_gather(ts, td, src2, dst2):
    mesh = plsc.VectorSubcoreMesh(core_axis_name="c", subcore_axis_name="s")
    fn = pl.kernel(
        _k2_body,
        out_type=(jax.ShapeDtypeStruct((EP, D), f32),
                  jax.ShapeDtypeStruct((EP, D), f32)),
        mesh=mesh,
        scratch_types=[pltpu.VMEM((RW, 128), i32),
                       pltpu.VMEM((4, 128, D), f32)]
                      + [pltpu.SemaphoreType.DMA] * 8,
    )
    return fn(ts, td, src2, dst2)


def _k3_body(gs, gd, ed_in,
             wh_ep, ws_ep, bs_ep, wv_ep, g_epln, b_epln, emb_e,
             g_ge, b_ge, wh_ge, ws_ge, bs_ge, wv_ge,
             m3, wb_m0, wd_m0, bs_m0, wv_m0,
             wh_m1, ws_m1, bs_m1, wv_m1,
             wh_m2, ws_m2, bs_m2, wv_m2,
             pay_out):
    ed = ed_in[...]
    n = ed.shape[0]
    ea, evs, etf = ed[:, 0:32], ed[:, 32:35], ed[:, 35:36]
    es, evs = _gvp1(ea, evs, wh_ep[0, 0], ws_ep[...], bs_ep[...], wv_ep[0, 0])
    es, evs = _ln_v(es, evs, 1, g_epln[...], b_epln[...])
    oh = (etf.astype(i32) == lax.broadcasted_iota(i32, (n, 4), 1)).astype(f32)
    es = jnp.concatenate([_dot(oh, emb_e[...]), es], axis=1)
    es, evs = _ln_v(es, evs, 1, g_ge[...], b_ge[...])
    es, evs = _gvp1(es, evs, wh_ge[0, 0], ws_ge[...], bs_ge[...], wv_ge[0, 0])

    gsr, gdr = gs[...], gd[...]
    vhs = gsr[:, 64:115] + gdr[:, 64:115] + _dot(evs, m3[...])
    vn = jnp.sqrt(jnp.maximum(_norms(vhs, 17), 1e-8))
    ms = (gsr[:, 0:64] + gdr[:, 0:64] + _dot(es, wb_m0[...])
          + _dot(vn, wd_m0[...]) + bs_m0[...])
    mvs = _dot(vhs, wv_m0[...])
    ms, mvs = _gvp_v(ms, mvs, 8, wh_m1[...], ws_m1[...], bs_m1[...],
                     wv_m1[...])
    ms, mvs = _gvp_v(ms, mvs, 8, wh_m2[...], ws_m2[...], bs_m2[...],
                     wv_m2[...])

    pay_out[...] = jnp.concatenate(
        [ms, mvs, jnp.ones((n, 1), f32), jnp.zeros((n, 39), f32)], axis=1)


def _messages(gs, gd, ed, w):
    B = 1600
    grid = E // B
    row = lambda width: pl.BlockSpec((B, width), lambda i: (i, 0))
    full = lambda a: pl.BlockSpec(a.shape, lambda i: (0,) * a.ndim)
    return pl.pallas_call(
        _k3_body,
        grid=(grid,),
        in_specs=[row(D), row(D), row(D)] + [full(a) for a in w],
        out_specs=row(D),
        out_shape=jax.ShapeDtypeStruct((EP, D), f32),
    )(gs, gd, ed, *w)


def _k4_body(pay_hbm, dst_hbm, agg_hbm, dst_v, pay_v, idx_v, zero_v, acc,
             *sems):
    lsem = sems[:4]
    csem = sems[4:]
    c = lax.axis_index("c")
    t = lax.axis_index("s")

    def zb(i, carry):
        for k in range(8):
            zero_v[i, 16 * k:16 * k + 16] = jnp.zeros((16,), f32)
        return carry

    lax.fori_loop(0, ZR, zb, 0)

    def fire(j, b):
        e0 = (t * TR + j) * 128
        pltpu.async_copy(pay_hbm.at[pl.ds(e0, 128)], pay_v.at[b], lsem[b])
        pltpu.async_copy(dst_hbm.at[pl.ds(e0, 128)], dst_v.at[b], lsem[b])

    def wait_l(b):
        pltpu.make_async_copy(pay_hbm.at[pl.ds(0, 128)], pay_v.at[b],
                              lsem[b]).wait()
        pltpu.make_async_copy(dst_hbm.at[pl.ds(0, 128)], dst_v.at[b],
                              lsem[b]).wait()

    def wait_c(b):
        pltpu.make_async_copy(pay_v.at[b], acc.at[idx_v.at[b]],
                              csem[b]).wait()

    for r in range(NR):

        @pl.when((r // 4) == c)
        def _(r=r):
            base = r * R
            for z in range(WT // ZR):
                pltpu.sync_copy(zero_v, acc.at[pl.ds(t * WT + z * ZR, ZR)])
            plsc.subcore_barrier()
            for b in range(2):
                fire(b, b)

            def body(jg, carry):
                for b in range(4):
                    j = jg * 4 + b
                    wait_l(b)
                    for k in range(8):
                        loc = dst_v[b, 16 * k:16 * k + 16] - base
                        loc = jnp.where((loc < 0) | (loc >= R), R, loc)
                        idx_v[b, 16 * k:16 * k + 16] = loc
                    pltpu.async_copy(pay_v.at[b], acc.at[idx_v.at[b]],
                                     csem[b], add=True)
                    jn = j + 2
                    bn = (b + 2) % 4

                    @pl.when(jn < TR)
                    def _():
                        @pl.when(j >= 2)
                        def _():
                            wait_c(bn)
                        fire(jn, bn)
                return carry

            lax.fori_loop(0, TR // 4, body, 0)
            for b in range(4):
                wait_c(b)
            plsc.subcore_barrier()
            pltpu.sync_copy(acc.at[pl.ds(t * WT, WT)],
                            agg_hbm.at[pl.ds(base + t * WT, WT)])
            plsc.subcore_barrier()


def _scatter(pay, dst1):
    mesh = plsc.VectorSubcoreMesh(core_axis_name="c", subcore_axis_name="s")
    fn = pl.kernel(
        _k4_body,
        out_type=jax.ShapeDtypeStruct((NP, D), f32),
        mesh=mesh,
        scratch_types=[pltpu.VMEM((4, 128), i32),
                       pltpu.VMEM((4, 128, D), f32),
                       pltpu.VMEM((4, 128), i32),
                       pltpu.VMEM((ZR, D), f32),
                       pltpu.VMEM_SHARED((R + 8, D), f32)]
                      + [pltpu.SemaphoreType.DMA] * 8,
    )
    return fn(pay, dst1)


def _k5_body(t_in, agg_in,
             g_n0, b_n0, wh_f0, ws_f0, bs_f0, wv_f0,
             wh_f1, ws_f1, bs_f1, wv_f1,
             g_n1, b_n1, g_fl, b_fl, wh_o, ws_o, bs_o, out):
    agg = agg_in[...]
    tr = t_in[...]
    s = tr[:, 0:64]
    vs = tr[:, 64:88]
    cnt = jnp.maximum(agg[:, 88:89], 1.0)
    s = s + agg[:, 0:64] / cnt
    vs = vs + agg[:, 64:88] / cnt
    s, vs = _ln_v(s, vs, 8, g_n0[...], b_n0[...])
    fs, fvs = _gvp_v(s, vs, 16, wh_f0[...], ws_f0[...], bs_f0[...],
                     wv_f0[...])
    fs, fvs = _gvp_v(fs, fvs, 16, wh_f1[...], ws_f1[...], bs_f1[...],
                     wv_f1[...])
    s, vs = _ln_v(s + fs, vs + fvs, 8, g_n1[...], b_n1[...])
    s, vs = _ln_v(s, vs, 8, g_fl[...], b_fl[...])
    o, _ = _gvp_v(s, vs, 8, wh_o[...], ws_o[...], bs_o[...], None)
    out[...] = o


def _node_post(t, agg, w):
    B = 1024
    grid = NP // B
    row = lambda width: pl.BlockSpec((B, width), lambda i: (i, 0))
    full = lambda a: pl.BlockSpec(a.shape, lambda i: (0,) * a.ndim)
    return pl.pallas_call(
        _k5_body,
        grid=(grid,),
        in_specs=[row(D), row(D)] + [full(a) for a in w],
        out_specs=row(8),
        out_shape=jax.ShapeDtypeStruct((NP, 8), f32),
    )(t, agg, *w)


def _pad_rows(a, rows):
    return jnp.pad(a, ((0, rows - a.shape[0]),) + ((0, 0),) * (a.ndim - 1))


def kernel(x_s, x_v, edge_index, ntypes, etypes, eattr_s, eattr_v, params):
    from jax.scipy.linalg import block_diag
    p = params
    r2 = lambda a: a.reshape(1, -1)
    bd3 = lambda a: block_diag(a, a, a)

    nin = _pad_rows(jnp.concatenate(
        [x_s.astype(f32), x_v.astype(f32).transpose(0, 2, 1).reshape(N, 9),
         ntypes.astype(f32).reshape(N, 1)], axis=1), NP)
    src2 = jnp.pad(edge_index[0].astype(i32), (0, EP - E),
                   constant_values=N).reshape(EP // 128, 128)
    dst1 = jnp.pad(edge_index[1].astype(i32), (0, EP - E), constant_values=N)
    dst2 = dst1.reshape(EP // 128, 128)
    ed = jnp.concatenate(
        [eattr_s.astype(f32), eattr_v.astype(f32).reshape(E, 3),
         etypes.astype(f32).reshape(E, 1), jnp.zeros((E, 92), f32)], axis=1)

    lp = p['convs'][0]
    wh0, ws0 = lp['msg0']['Wh'], lp['msg0']['Ws']
    w1 = [bd3(p['np_gvp']['Wh']), p['np_gvp']['Ws'], r2(p['np_gvp']['bs']),
          bd3(p['np_gvp']['Wv']), r2(p['np_ln']['g']), r2(p['np_ln']['b']),
          p['ntype_emb'], r2(p['gn_ln']['g']), r2(p['gn_ln']['b']),
          bd3(p['gn_gvp']['Wh']), p['gn_gvp']['Ws'], r2(p['gn_gvp']['bs']),
          bd3(p['gn_gvp']['Wv']),
          bd3(wh0[0:8]), bd3(wh0[9:17]), ws0[0:64], ws0[96:160]]
    t, ts, td = _node_pre(nin, w1)

    gs, gd = _edge_gather(ts, td, src2, dst2)

    w3 = [p['ep_gvp']['Wh'], p['ep_gvp']['Ws'], r2(p['ep_gvp']['bs']),
          p['ep_gvp']['Wv'], r2(p['ep_ln']['g']), r2(p['ep_ln']['b']),
          p['etype_emb'], r2(p['ge_ln']['g']), r2(p['ge_ln']['b']),
          p['ge_gvp']['Wh'], p['ge_gvp']['Ws'], r2(p['ge_gvp']['bs']),
          p['ge_gvp']['Wv'],
          bd3(wh0[8:9]), ws0[64:96], ws0[160:177], r2(lp['msg0']['bs']),
          bd3(lp['msg0']['Wv']),
          bd3(lp['msg1']['Wh']), lp['msg1']['Ws'], r2(lp['msg1']['bs']),
          bd3(lp['msg1']['Wv']),
          bd3(lp['msg2']['Wh']), lp['msg2']['Ws'], r2(lp['msg2']['bs']),
          bd3(lp['msg2']['Wv'])]
    pay = _messages(gs, gd, ed, w3)

    agg = _scatter(pay, dst1)

    w5 = [r2(lp['norm0']['g']), r2(lp['norm0']['b']),
          bd3(lp['ff0']['Wh']), lp['ff0']['Ws'], r2(lp['ff0']['bs']),
          bd3(lp['ff0']['Wv']),
          bd3(lp['ff1']['Wh']), lp['ff1']['Ws'], r2(lp['ff1']['bs']),
          bd3(lp['ff1']['Wv']),
          r2(lp['norm1']['g']), r2(lp['norm1']['b']),
          r2(p['final_ln']['g']), r2(p['final_ln']['b']),
          bd3(p['out_gvp']['Wh']), p['out_gvp']['Ws'], r2(p['out_gvp']['bs'])]
    out = _node_post(t, agg, w5)
    return out[:N]


# speedup vs baseline: 1.0425x; 1.0018x over previous
import functools

import jax
import jax.numpy as jnp
from jax import lax
from jax.experimental import pallas as pl
from jax.experimental.pallas import tpu as pltpu
from jax.experimental.pallas import tpu_sc as plsc

N = 50000
NP = 50176
E = 800000
EP = 819200
D = 128
NR = 8
R = NP // NR
NC = 2
NS = 16
NW = NC * NS
RW = EP // (NW * 128)
TR = (EP // 128) // NS
WT = R // NS
ZR = 56
NB = 4

f32 = jnp.float32
i32 = jnp.int32


def _dot(a, b):
    return jnp.dot(a, b, preferred_element_type=f32)


def _norms(vhs, H):
    return (vhs[:, 0:H] * vhs[:, 0:H] + vhs[:, H:2 * H] * vhs[:, H:2 * H]
            + vhs[:, 2 * H:3 * H] * vhs[:, 2 * H:3 * H])


def _gvp_v(s, vs, H, whb, ws, bs, wvb):
    vhs = _dot(vs, whb)
    vn = jnp.sqrt(jnp.maximum(_norms(vhs, H), 1e-8))
    so = _dot(jnp.concatenate([s, vn], axis=1), ws) + bs
    vo = None if wvb is None else _dot(vhs, wvb)
    return so, vo


def _gvp1(s, vs, wh, ws, bs, wv):
    vhs = vs * wh
    vn = jnp.sqrt(jnp.maximum(_norms(vhs, 1), 1e-8))
    so = _dot(jnp.concatenate([s, vn], axis=1), ws) + bs
    return so, vhs * wv


def _ln_v(s, vs, K, g, b):
    mu = jnp.mean(s, axis=1, keepdims=True)
    var = jnp.mean((s - mu) * (s - mu), axis=1, keepdims=True)
    so = (s - mu) / jnp.sqrt(var + 1e-5) * g + b
    nk = jnp.maximum(_norms(vs, K), 1e-8)
    rms = jnp.sqrt(jnp.mean(nk, axis=1, keepdims=True))
    return so, vs / rms


def _k1_body(nin, wh_np, ws_np, bs_np, wv_np, g_npln, b_npln, emb_n,
             g_gn, b_gn, wh_gn, ws_gn, bs_gn, wv_gn,
             wha, whc, wsa, wsc, t_out, ts_out, td_out):
    r = nin[...]
    n = r.shape[0]
    xs, vs, ntf = r[:, 0:6], r[:, 6:15], r[:, 15:16]
    s, vs = _gvp_v(xs, vs, 8, wh_np[...], ws_np[...], bs_np[...], wv_np[...])
    s, vs = _ln_v(s, vs, 8, g_npln[...], b_npln[...])
    oh = (ntf.astype(i32) == lax.broadcasted_iota(i32, (n, 20), 1)).astype(f32)
    s = jnp.concatenate([_dot(oh, emb_n[...]), s], axis=1)
    s, vs = _ln_v(s, vs, 8, g_gn[...], b_gn[...])
    s, vs = _gvp_v(s, vs, 8, wh_gn[...], ws_gn[...], bs_gn[...], wv_gn[...])
    z = jnp.zeros((n, 13), f32)
    t_out[...] = jnp.concatenate([s, vs, jnp.zeros((n, 40), f32)], axis=1)
    ts_out[...] = jnp.concatenate(
        [_dot(s, wsa[...]), _dot(vs, wha[...]), z], axis=1)
    td_out[...] = jnp.concatenate(
        [_dot(s, wsc[...]), _dot(vs, whc[...]), z], axis=1)


def _node_pre(nin, w):
    B = 1024
    grid = NP // B
    row = lambda width: pl.BlockSpec((B, width), lambda i: (i, 0))
    full = lambda a: pl.BlockSpec(a.shape, lambda i: (0,) * a.ndim)
    return pl.pallas_call(
        _k1_body,
        grid=(grid,),
        in_specs=[row(16)] + [full(a) for a in w],
        out_specs=[row(D)] * 3,
        out_shape=[jax.ShapeDtypeStruct((NP, D), f32)] * 3,
    )(nin, *w)


def _k2_body(ts_hbm, td_hbm, src_hbm, dst_hbm, gs_hbm, gd_hbm, idx_v, rows_v,
             *sems):
    gsem = sems[:4]
    ssem = sems[4:]
    wid = lax.axis_index("s") * NC + lax.axis_index("c")
    for t_hbm, idx_hbm, out_hbm in ((ts_hbm, src_hbm, gs_hbm),
                                    (td_hbm, dst_hbm, gd_hbm)):
        pltpu.sync_copy(idx_hbm.at[pl.ds(wid * RW, RW)], idx_v)

        def fire(j, b):
            pltpu.async_copy(t_hbm.at[idx_v.at[j]], rows_v.at[b], gsem[b])

        def store(b, j):
            pltpu.async_copy(rows_v.at[b],
                             out_hbm.at[pl.ds((wid * RW + j) * 128, 128)],
                             ssem[b])

        def wait_g(b):
            pltpu.make_async_copy(t_hbm.at[idx_v.at[0]], rows_v.at[b],
                                  gsem[b]).wait()

        def wait_s(b):
            pltpu.make_async_copy(rows_v.at[b],
                                  out_hbm.at[pl.ds(0, 128)], ssem[b]).wait()

        for b in range(2):
            fire(b, b)

        def body(jg, carry):
            for b in range(4):
                j = jg * 4 + b
                wait_g(b)
                store(b, j)
                jn = j + 2
                bn = (b + 2) % 4

                @pl.when(jn < RW)
                def _():
                    @pl.when(j >= 2)
                    def _():
                        wait_s(bn)
                    fire(jn, bn)
            return carry

        lax.fori_loop(0, RW // 4, body, 0)
        for b in range(4):
            wait_s(b)


def _edge_gather(ts, td, src2, dst2):
    mesh = plsc.VectorSubcoreMesh(core_axis_name="c", subcore_axis_name="s")
    fn = pl.kernel(
        _k2_body,
        out_type=(jax.ShapeDtypeStruct((EP, D), f32),
                  jax.ShapeDtypeStruct((EP, D), f32)),
        mesh=mesh,
        scratch_types=[pltpu.VMEM((RW, 128), i32),
                       pltpu.VMEM((4, 128, D), f32)]
                      + [pltpu.SemaphoreType.DMA] * 8,
        compiler_params=pltpu.CompilerParams(use_tc_tiling_on_sc=True),
    )
    return fn(ts, td, src2, dst2)


def _k3_body(gs, gd, ed_in,
             wh_ep, ws_ep, bs_ep, wv_ep, g_epln, b_epln, emb_e,
             g_ge, b_ge, wh_ge, ws_ge, bs_ge, wv_ge,
             m3, wb_m0, wd_m0, bs_m0, wv_m0,
             wh_m1, ws_m1, bs_m1, wv_m1,
             wh_m2, ws_m2, bs_m2, wv_m2,
             pay_out):
    ed = ed_in[...]
    n = ed.shape[0]
    ea, evs, etf = ed[:, 0:32], ed[:, 32:35], ed[:, 35:36]
    es, evs = _gvp1(ea, evs, wh_ep[0, 0], ws_ep[...], bs_ep[...], wv_ep[0, 0])
    es, evs = _ln_v(es, evs, 1, g_epln[...], b_epln[...])
    oh = (etf.astype(i32) == lax.broadcasted_iota(i32, (n, 4), 1)).astype(f32)
    es = jnp.concatenate([_dot(oh, emb_e[...]), es], axis=1)
    es, evs = _ln_v(es, evs, 1, g_ge[...], b_ge[...])
    es, evs = _gvp1(es, evs, wh_ge[0, 0], ws_ge[...], bs_ge[...], wv_ge[0, 0])

    gsr, gdr = gs[...], gd[...]
    vhs = gsr[:, 64:115] + gdr[:, 64:115] + _dot(evs, m3[...])
    vn = jnp.sqrt(jnp.maximum(_norms(vhs, 17), 1e-8))
    ms = (gsr[:, 0:64] + gdr[:, 0:64] + _dot(es, wb_m0[...])
          + _dot(vn, wd_m0[...]) + bs_m0[...])
    mvs = _dot(vhs, wv_m0[...])
    ms, mvs = _gvp_v(ms, mvs, 8, wh_m1[...], ws_m1[...], bs_m1[...],
                     wv_m1[...])
    ms, mvs = _gvp_v(ms, mvs, 8, wh_m2[...], ws_m2[...], bs_m2[...],
                     wv_m2[...])

    pay_out[...] = jnp.concatenate(
        [ms, mvs, jnp.ones((n, 1), f32), jnp.zeros((n, 39), f32)], axis=1)


def _messages(gs, gd, ed, w):
    B = 1600
    grid = E // B
    row = lambda width: pl.BlockSpec((B, width), lambda i: (i, 0))
    full = lambda a: pl.BlockSpec(a.shape, lambda i: (0,) * a.ndim)
    return pl.pallas_call(
        _k3_body,
        grid=(grid,),
        in_specs=[row(D), row(D), row(D)] + [full(a) for a in w],
        out_specs=row(D),
        out_shape=jax.ShapeDtypeStruct((EP, D), f32),
    )(gs, gd, ed, *w)


def _k4_body(pay_hbm, dst_hbm, agg_hbm, dst_v, pay_v, idx_v, zero_v, acc,
             *sems):
    lsem = sems[:4]
    csem = sems[4:]
    c = lax.axis_index("c")
    t = lax.axis_index("s")

    def zb(i, carry):
        for k in range(8):
            zero_v[i, 16 * k:16 * k + 16] = jnp.zeros((16,), f32)
        return carry

    lax.fori_loop(0, ZR, zb, 0)

    def fire(j, b):
        e0 = (t * TR + j) * 128
        pltpu.async_copy(pay_hbm.at[pl.ds(e0, 128)], pay_v.at[b], lsem[b])
        pltpu.async_copy(dst_hbm.at[pl.ds(e0, 128)], dst_v.at[b], lsem[b])

    def wait_l(b):
        pltpu.make_async_copy(pay_hbm.at[pl.ds(0, 128)], pay_v.at[b],
                              lsem[b]).wait()
        pltpu.make_async_copy(dst_hbm.at[pl.ds(0, 128)], dst_v.at[b],
                              lsem[b]).wait()

    def wait_c(b):
        pltpu.make_async_copy(pay_v.at[b], acc.at[idx_v.at[b]],
                              csem[b]).wait()

    for r in range(NR):

        @pl.when((r // 4) == c)
        def _(r=r):
            base = r * R
            for z in range(WT // ZR):
                pltpu.sync_copy(zero_v, acc.at[pl.ds(t * WT + z * ZR, ZR)])
            plsc.subcore_barrier()
            for b in range(2):
                fire(b, b)

            def body(jg, carry):
                for b in range(4):
                    j = jg * 4 + b
                    wait_l(b)
                    for k in range(8):
                        loc = dst_v[b, 16 * k:16 * k + 16] - base
                        loc = jnp.where((loc < 0) | (loc >= R), R, loc)
                        idx_v[b, 16 * k:16 * k + 16] = loc
                    pltpu.async_copy(pay_v.at[b], acc.at[idx_v.at[b]],
                                     csem[b], add=True)
                    jn = j + 2
                    bn = (b + 2) % 4

                    @pl.when(jn < TR)
                    def _():
                        @pl.when(j >= 2)
                        def _():
                            wait_c(bn)
                        fire(jn, bn)
                return carry

            lax.fori_loop(0, TR // 4, body, 0)
            for b in range(4):
                wait_c(b)
            plsc.subcore_barrier()
            pltpu.sync_copy(acc.at[pl.ds(t * WT, WT)],
                            agg_hbm.at[pl.ds(base + t * WT, WT)])
            plsc.subcore_barrier()


def _scatter(pay, dst1):
    mesh = plsc.VectorSubcoreMesh(core_axis_name="c", subcore_axis_name="s")
    fn = pl.kernel(
        _k4_body,
        out_type=jax.ShapeDtypeStruct((NP, D), f32),
        mesh=mesh,
        scratch_types=[pltpu.VMEM((4, 128), i32),
                       pltpu.VMEM((4, 128, D), f32),
                       pltpu.VMEM((4, 128), i32),
                       pltpu.VMEM((ZR, D), f32),
                       pltpu.VMEM_SHARED((R + 8, D), f32)]
                      + [pltpu.SemaphoreType.DMA] * 8,
        compiler_params=pltpu.CompilerParams(use_tc_tiling_on_sc=True),
    )
    return fn(pay, dst1)


def _k5_body(t_in, agg_in,
             g_n0, b_n0, wh_f0, ws_f0, bs_f0, wv_f0,
             wh_f1, ws_f1, bs_f1, wv_f1,
             g_n1, b_n1, g_fl, b_fl, wh_o, ws_o, bs_o, out):
    agg = agg_in[...]
    tr = t_in[...]
    s = tr[:, 0:64]
    vs = tr[:, 64:88]
    cnt = jnp.maximum(agg[:, 88:89], 1.0)
    s = s + agg[:, 0:64] / cnt
    vs = vs + agg[:, 64:88] / cnt
    s, vs = _ln_v(s, vs, 8, g_n0[...], b_n0[...])
    fs, fvs = _gvp_v(s, vs, 16, wh_f0[...], ws_f0[...], bs_f0[...],
                     wv_f0[...])
    fs, fvs = _gvp_v(fs, fvs, 16, wh_f1[...], ws_f1[...], bs_f1[...],
                     wv_f1[...])
    s, vs = _ln_v(s + fs, vs + fvs, 8, g_n1[...], b_n1[...])
    s, vs = _ln_v(s, vs, 8, g_fl[...], b_fl[...])
    o, _ = _gvp_v(s, vs, 8, wh_o[...], ws_o[...], bs_o[...], None)
    out[...] = o


def _node_post(t, agg, w):
    B = 1024
    grid = NP // B
    row = lambda width: pl.BlockSpec((B, width), lambda i: (i, 0))
    full = lambda a: pl.BlockSpec(a.shape, lambda i: (0,) * a.ndim)
    return pl.pallas_call(
        _k5_body,
        grid=(grid,),
        in_specs=[row(D), row(D)] + [full(a) for a in w],
        out_specs=row(8),
        out_shape=jax.ShapeDtypeStruct((NP, 8), f32),
    )(t, agg, *w)


def _pad_rows(a, rows):
    return jnp.pad(a, ((0, rows - a.shape[0]),) + ((0, 0),) * (a.ndim - 1))


def kernel(x_s, x_v, edge_index, ntypes, etypes, eattr_s, eattr_v, params):
    from jax.scipy.linalg import block_diag
    p = params
    r2 = lambda a: a.reshape(1, -1)
    bd3 = lambda a: block_diag(a, a, a)

    nin = _pad_rows(jnp.concatenate(
        [x_s.astype(f32), x_v.astype(f32).transpose(0, 2, 1).reshape(N, 9),
         ntypes.astype(f32).reshape(N, 1)], axis=1), NP)
    src2 = jnp.pad(edge_index[0].astype(i32), (0, EP - E),
                   constant_values=N).reshape(EP // 128, 128)
    dst1 = jnp.pad(edge_index[1].astype(i32), (0, EP - E), constant_values=N)
    dst2 = dst1.reshape(EP // 128, 128)
    ed = jnp.concatenate(
        [eattr_s.astype(f32), eattr_v.astype(f32).reshape(E, 3),
         etypes.astype(f32).reshape(E, 1), jnp.zeros((E, 92), f32)], axis=1)

    lp = p['convs'][0]
    wh0, ws0 = lp['msg0']['Wh'], lp['msg0']['Ws']
    w1 = [bd3(p['np_gvp']['Wh']), p['np_gvp']['Ws'], r2(p['np_gvp']['bs']),
          bd3(p['np_gvp']['Wv']), r2(p['np_ln']['g']), r2(p['np_ln']['b']),
          p['ntype_emb'], r2(p['gn_ln']['g']), r2(p['gn_ln']['b']),
          bd3(p['gn_gvp']['Wh']), p['gn_gvp']['Ws'], r2(p['gn_gvp']['bs']),
          bd3(p['gn_gvp']['Wv']),
          bd3(wh0[0:8]), bd3(wh0[9:17]), ws0[0:64], ws0[96:160]]
    t, ts, td = _node_pre(nin, w1)

    gs, gd = _edge_gather(ts, td, src2, dst2)

    w3 = [p['ep_gvp']['Wh'], p['ep_gvp']['Ws'], r2(p['ep_gvp']['bs']),
          p['ep_gvp']['Wv'], r2(p['ep_ln']['g']), r2(p['ep_ln']['b']),
          p['etype_emb'], r2(p['ge_ln']['g']), r2(p['ge_ln']['b']),
          p['ge_gvp']['Wh'], p['ge_gvp']['Ws'], r2(p['ge_gvp']['bs']),
          p['ge_gvp']['Wv'],
          bd3(wh0[8:9]), ws0[64:96], ws0[160:177], r2(lp['msg0']['bs']),
          bd3(lp['msg0']['Wv']),
          bd3(lp['msg1']['Wh']), lp['msg1']['Ws'], r2(lp['msg1']['bs']),
          bd3(lp['msg1']['Wv']),
          bd3(lp['msg2']['Wh']), lp['msg2']['Ws'], r2(lp['msg2']['bs']),
          bd3(lp['msg2']['Wv'])]
    pay = _messages(gs, gd, ed, w3)

    agg = _scatter(pay, dst1)

    w5 = [r2(lp['norm0']['g']), r2(lp['norm0']['b']),
          bd3(lp['ff0']['Wh']), lp['ff0']['Ws'], r2(lp['ff0']['bs']),
          bd3(lp['ff0']['Wv']),
          bd3(lp['ff1']['Wh']), lp['ff1']['Ws'], r2(lp['ff1']['bs']),
          bd3(lp['ff1']['Wv']),
          r2(lp['norm1']['g']), r2(lp['norm1']['b']),
          r2(p['final_ln']['g']), r2(p['final_ln']['b']),
          bd3(p['out_gvp']['Wh']), p['out_gvp']['Ws'], r2(p['out_gvp']['bs'])]
    out = _node_post(t, agg, w5)
    return out[:N]


# speedup vs baseline: 1.2616x; 1.2102x over previous
import functools

import jax
import jax.numpy as jnp
from jax import lax
from jax.experimental import pallas as pl
from jax.experimental.pallas import tpu as pltpu
from jax.experimental.pallas import tpu_sc as plsc

N = 50000
NP = 50176
E = 800000
EP = 819200
D = 128
NR = 8
R = NP // NR
NC = 2
NS = 16
NW = NC * NS
RW = EP // (NW * 128)
TR = (EP // 128) // NS
WT = R // NS
ZR = 56
NB = 4

f32 = jnp.float32
i32 = jnp.int32


def _dot(a, b):
    return jnp.dot(a, b, preferred_element_type=f32)


def _norms(vhs, H):
    return (vhs[:, 0:H] * vhs[:, 0:H] + vhs[:, H:2 * H] * vhs[:, H:2 * H]
            + vhs[:, 2 * H:3 * H] * vhs[:, 2 * H:3 * H])


def _gvp_v(s, vs, H, whb, ws, bs, wvb):
    vhs = _dot(vs, whb)
    vn = jnp.sqrt(jnp.maximum(_norms(vhs, H), 1e-8))
    so = _dot(jnp.concatenate([s, vn], axis=1), ws) + bs
    vo = None if wvb is None else _dot(vhs, wvb)
    return so, vo


def _gvp1(s, vs, wh, ws, bs, wv):
    vhs = vs * wh
    vn = jnp.sqrt(jnp.maximum(_norms(vhs, 1), 1e-8))
    so = _dot(jnp.concatenate([s, vn], axis=1), ws) + bs
    return so, vhs * wv


def _ln_v(s, vs, K, g, b):
    mu = jnp.mean(s, axis=1, keepdims=True)
    var = jnp.mean((s - mu) * (s - mu), axis=1, keepdims=True)
    so = (s - mu) / jnp.sqrt(var + 1e-5) * g + b
    nk = jnp.maximum(_norms(vs, K), 1e-8)
    rms = jnp.sqrt(jnp.mean(nk, axis=1, keepdims=True))
    return so, vs / rms


def _k1_body(nin, wh_np, ws_np, bs_np, wv_np, g_npln, b_npln, emb_n,
             g_gn, b_gn, wh_gn, ws_gn, bs_gn, wv_gn,
             wha, whc, wsa, wsc, t_out, ts_out, td_out):
    r = nin[...]
    n = r.shape[0]
    xs, vs, ntf = r[:, 0:6], r[:, 6:15], r[:, 15:16]
    s, vs = _gvp_v(xs, vs, 8, wh_np[...], ws_np[...], bs_np[...], wv_np[...])
    s, vs = _ln_v(s, vs, 8, g_npln[...], b_npln[...])
    oh = (ntf.astype(i32) == lax.broadcasted_iota(i32, (n, 20), 1)).astype(f32)
    s = jnp.concatenate([_dot(oh, emb_n[...]), s], axis=1)
    s, vs = _ln_v(s, vs, 8, g_gn[...], b_gn[...])
    s, vs = _gvp_v(s, vs, 8, wh_gn[...], ws_gn[...], bs_gn[...], wv_gn[...])
    z = jnp.zeros((n, 13), f32)
    t_out[...] = jnp.concatenate([s, vs, jnp.zeros((n, 40), f32)], axis=1)
    ts_out[...] = jnp.concatenate(
        [_dot(s, wsa[...]), _dot(vs, wha[...]), z], axis=1)
    td_out[...] = jnp.concatenate(
        [_dot(s, wsc[...]), _dot(vs, whc[...]), z], axis=1)


def _node_pre(nin, w):
    B = 1024
    grid = NP // B
    row = lambda width: pl.BlockSpec((B, width), lambda i: (i, 0))
    full = lambda a: pl.BlockSpec(a.shape, lambda i: (0,) * a.ndim)
    return pl.pallas_call(
        _k1_body,
        grid=(grid,),
        in_specs=[row(16)] + [full(a) for a in w],
        out_specs=[row(D)] * 3,
        out_shape=[jax.ShapeDtypeStruct((NP, D), f32)] * 3,
    )(nin, *w)


def _edge_gather(ts, td, src1, dst1):
    epl = src1.shape[0]
    rw = epl // (NW * 128)

    def body(ts_hbm, td_hbm, src_hbm, dst_hbm, gs_hbm, gd_hbm, idx_v, rows_v,
             *sems):
        gsem = sems[:4]
        ssem = sems[4:]
        wid = lax.axis_index("s") * NC + lax.axis_index("c")
        for t_hbm, idx_hbm, out_hbm in ((ts_hbm, src_hbm, gs_hbm),
                                        (td_hbm, dst_hbm, gd_hbm)):
            pltpu.sync_copy(idx_hbm.at[pl.ds(wid * rw * 128, rw * 128)],
                            idx_v)

            def fire(j, b):
                pltpu.async_copy(t_hbm.at[idx_v.at[pl.ds(j * 128, 128)]],
                                 rows_v.at[b], gsem[b])

            def store(b, j):
                pltpu.async_copy(rows_v.at[b],
                                 out_hbm.at[pl.ds((wid * rw + j) * 128, 128)],
                                 ssem[b])

            def wait_g(b):
                pltpu.make_async_copy(t_hbm.at[idx_v.at[pl.ds(0, 128)]],
                                      rows_v.at[b], gsem[b]).wait()

            def wait_s(b):
                pltpu.make_async_copy(rows_v.at[b],
                                      out_hbm.at[pl.ds(0, 128)],
                                      ssem[b]).wait()

            for b in range(2):
                fire(b, b)

            def loop(jg, carry):
                for b in range(4):
                    j = jg * 4 + b
                    wait_g(b)
                    store(b, j)
                    jn = j + 2
                    bn = (b + 2) % 4

                    @pl.when(jn < rw)
                    def _():
                        @pl.when(j >= 2)
                        def _():
                            wait_s(bn)
                        fire(jn, bn)
                return carry

            lax.fori_loop(0, rw // 4, loop, 0)
            for b in range(4):
                wait_s(b)

    mesh = plsc.VectorSubcoreMesh(core_axis_name="c", subcore_axis_name="s")
    fn = pl.kernel(
        body,
        out_type=(jax.ShapeDtypeStruct((epl, D), f32),
                  jax.ShapeDtypeStruct((epl, D), f32)),
        mesh=mesh,
        scratch_types=[pltpu.VMEM((rw * 128,), i32),
                       pltpu.VMEM((4, 128, D), f32)]
                      + [pltpu.SemaphoreType.DMA] * 8,
        compiler_params=pltpu.CompilerParams(use_tc_tiling_on_sc=True),
    )
    return fn(ts, td, src1, dst1)


def _k3_body(gs, gd, ed_in,
             wh_ep, ws_ep, bs_ep, wv_ep, g_epln, b_epln, emb_e,
             g_ge, b_ge, wh_ge, ws_ge, bs_ge, wv_ge,
             m3, wb_m0, wd_m0, bs_m0, wv_m0,
             wh_m1, ws_m1, bs_m1, wv_m1,
             wh_m2, ws_m2, bs_m2, wv_m2,
             pay_out):
    ed = ed_in[...]
    n = ed.shape[0]
    ea, evs, etf = ed[:, 0:32], ed[:, 32:35], ed[:, 35:36]
    es, evs = _gvp1(ea, evs, wh_ep[0, 0], ws_ep[...], bs_ep[...], wv_ep[0, 0])
    es, evs = _ln_v(es, evs, 1, g_epln[...], b_epln[...])
    oh = (etf.astype(i32) == lax.broadcasted_iota(i32, (n, 4), 1)).astype(f32)
    es = jnp.concatenate([_dot(oh, emb_e[...]), es], axis=1)
    es, evs = _ln_v(es, evs, 1, g_ge[...], b_ge[...])
    es, evs = _gvp1(es, evs, wh_ge[0, 0], ws_ge[...], bs_ge[...], wv_ge[0, 0])

    gsr, gdr = gs[...], gd[...]
    vhs = gsr[:, 64:115] + gdr[:, 64:115] + _dot(evs, m3[...])
    vn = jnp.sqrt(jnp.maximum(_norms(vhs, 17), 1e-8))
    ms = (gsr[:, 0:64] + gdr[:, 0:64] + _dot(es, wb_m0[...])
          + _dot(vn, wd_m0[...]) + bs_m0[...])
    mvs = _dot(vhs, wv_m0[...])
    ms, mvs = _gvp_v(ms, mvs, 8, wh_m1[...], ws_m1[...], bs_m1[...],
                     wv_m1[...])
    ms, mvs = _gvp_v(ms, mvs, 8, wh_m2[...], ws_m2[...], bs_m2[...],
                     wv_m2[...])

    pay_out[...] = jnp.concatenate(
        [ms, mvs, jnp.ones((n, 1), f32), jnp.zeros((n, 39), f32)], axis=1)


def _messages(gs, gd, ed, w):
    B = 1600
    grid = ed.shape[0] // B
    row = lambda width: pl.BlockSpec((B, width), lambda i: (i, 0))
    full = lambda a: pl.BlockSpec(a.shape, lambda i: (0,) * a.ndim)
    return pl.pallas_call(
        _k3_body,
        grid=(grid,),
        in_specs=[row(D), row(D), row(D)] + [full(a) for a in w],
        out_specs=row(D),
        out_shape=jax.ShapeDtypeStruct((gs.shape[0], D), f32),
    )(gs, gd, ed, *w)


def _scatter(pay, dst1):
    tr = (pay.shape[0] // 128) // NS

    def body(pay_hbm, dst_hbm, agg_hbm, dst_v, pay_v, idx_v, zero_v, acc,
             *sems):
        lsem = sems[:4]
        csem = sems[4:]
        c = lax.axis_index("c")
        t = lax.axis_index("s")

        def zb(i, carry):
            for k in range(8):
                zero_v[i, 16 * k:16 * k + 16] = jnp.zeros((16,), f32)
            return carry

        lax.fori_loop(0, ZR, zb, 0)

        def fire(j, b):
            e0 = (t * tr + j) * 128
            pltpu.async_copy(pay_hbm.at[pl.ds(e0, 128)], pay_v.at[b], lsem[b])
            pltpu.async_copy(dst_hbm.at[pl.ds(e0, 128)], dst_v.at[b], lsem[b])

        def wait_l(b):
            pltpu.make_async_copy(pay_hbm.at[pl.ds(0, 128)], pay_v.at[b],
                                  lsem[b]).wait()
            pltpu.make_async_copy(dst_hbm.at[pl.ds(0, 128)], dst_v.at[b],
                                  lsem[b]).wait()

        def wait_c(b):
            pltpu.make_async_copy(pay_v.at[b], acc.at[idx_v.at[b]],
                                  csem[b]).wait()

        for r in range(NR):

            @pl.when((r // 4) == c)
            def _(r=r):
                base = r * R
                for z in range(WT // ZR):
                    pltpu.sync_copy(zero_v, acc.at[pl.ds(t * WT + z * ZR, ZR)])
                plsc.subcore_barrier()
                for b in range(2):
                    fire(b, b)

                def loop(jg, carry):
                    for b in range(4):
                        j = jg * 4 + b
                        wait_l(b)
                        for k in range(8):
                            loc = dst_v[b, 16 * k:16 * k + 16] - base
                            loc = jnp.where((loc < 0) | (loc >= R), R, loc)
                            idx_v[b, 16 * k:16 * k + 16] = loc
                        pltpu.async_copy(pay_v.at[b], acc.at[idx_v.at[b]],
                                         csem[b], add=True)
                        jn = j + 2
                        bn = (b + 2) % 4

                        @pl.when(jn < tr)
                        def _():
                            @pl.when(j >= 2)
                            def _():
                                wait_c(bn)
                            fire(jn, bn)
                    return carry

                lax.fori_loop(0, tr // 4, loop, 0)
                for b in range(4):
                    wait_c(b)
                plsc.subcore_barrier()
                pltpu.sync_copy(acc.at[pl.ds(t * WT, WT)],
                                agg_hbm.at[pl.ds(base + t * WT, WT)])
                plsc.subcore_barrier()

    mesh = plsc.VectorSubcoreMesh(core_axis_name="c", subcore_axis_name="s")
    fn = pl.kernel(
        body,
        out_type=jax.ShapeDtypeStruct((NP, D), f32),
        mesh=mesh,
        scratch_types=[pltpu.VMEM((4, 128), i32),
                       pltpu.VMEM((4, 128, D), f32),
                       pltpu.VMEM((4, 128), i32),
                       pltpu.VMEM((ZR, D), f32),
                       pltpu.VMEM_SHARED((R + 8, D), f32)]
                      + [pltpu.SemaphoreType.DMA] * 8,
        compiler_params=pltpu.CompilerParams(use_tc_tiling_on_sc=True),
    )
    return fn(pay, dst1)


def _k5_body(t_in, a1_in, a2_in,
             g_n0, b_n0, wh_f0, ws_f0, bs_f0, wv_f0,
             wh_f1, ws_f1, bs_f1, wv_f1,
             g_n1, b_n1, g_fl, b_fl, wh_o, ws_o, bs_o, out):
    agg = a1_in[...] + a2_in[...]
    tr = t_in[...]
    s = tr[:, 0:64]
    vs = tr[:, 64:88]
    cnt = jnp.maximum(agg[:, 88:89], 1.0)
    s = s + agg[:, 0:64] / cnt
    vs = vs + agg[:, 64:88] / cnt
    s, vs = _ln_v(s, vs, 8, g_n0[...], b_n0[...])
    fs, fvs = _gvp_v(s, vs, 16, wh_f0[...], ws_f0[...], bs_f0[...],
                     wv_f0[...])
    fs, fvs = _gvp_v(fs, fvs, 16, wh_f1[...], ws_f1[...], bs_f1[...],
                     wv_f1[...])
    s, vs = _ln_v(s + fs, vs + fvs, 8, g_n1[...], b_n1[...])
    s, vs = _ln_v(s, vs, 8, g_fl[...], b_fl[...])
    o, _ = _gvp_v(s, vs, 8, wh_o[...], ws_o[...], bs_o[...], None)
    out[...] = o


def _node_post(t, a1, a2, w):
    B = 1024
    grid = NP // B
    row = lambda width: pl.BlockSpec((B, width), lambda i: (i, 0))
    full = lambda a: pl.BlockSpec(a.shape, lambda i: (0,) * a.ndim)
    return pl.pallas_call(
        _k5_body,
        grid=(grid,),
        in_specs=[row(D), row(D), row(D)] + [full(a) for a in w],
        out_specs=row(8),
        out_shape=jax.ShapeDtypeStruct((NP, 8), f32),
    )(t, a1, a2, *w)


def _pad_rows(a, rows):
    return jnp.pad(a, ((0, rows - a.shape[0]),) + ((0, 0),) * (a.ndim - 1))


def kernel(x_s, x_v, edge_index, ntypes, etypes, eattr_s, eattr_v, params):
    from jax.scipy.linalg import block_diag
    p = params
    r2 = lambda a: a.reshape(1, -1)
    bd3 = lambda a: block_diag(a, a, a)

    nin = _pad_rows(jnp.concatenate(
        [x_s.astype(f32), x_v.astype(f32).transpose(0, 2, 1).reshape(N, 9),
         ntypes.astype(f32).reshape(N, 1)], axis=1), NP)
    src1 = jnp.pad(edge_index[0].astype(i32), (0, EP - E), constant_values=N)
    dst1 = jnp.pad(edge_index[1].astype(i32), (0, EP - E), constant_values=N)
    ed = jnp.concatenate(
        [eattr_s.astype(f32), eattr_v.astype(f32).reshape(E, 3),
         etypes.astype(f32).reshape(E, 1), jnp.zeros((E, 92), f32)], axis=1)

    lp = p['convs'][0]
    wh0, ws0 = lp['msg0']['Wh'], lp['msg0']['Ws']
    w1 = [bd3(p['np_gvp']['Wh']), p['np_gvp']['Ws'], r2(p['np_gvp']['bs']),
          bd3(p['np_gvp']['Wv']), r2(p['np_ln']['g']), r2(p['np_ln']['b']),
          p['ntype_emb'], r2(p['gn_ln']['g']), r2(p['gn_ln']['b']),
          bd3(p['gn_gvp']['Wh']), p['gn_gvp']['Ws'], r2(p['gn_gvp']['bs']),
          bd3(p['gn_gvp']['Wv']),
          bd3(wh0[0:8]), bd3(wh0[9:17]), ws0[0:64], ws0[96:160]]
    t, ts, td = _node_pre(nin, w1)

    w3 = [p['ep_gvp']['Wh'], p['ep_gvp']['Ws'], r2(p['ep_gvp']['bs']),
          p['ep_gvp']['Wv'], r2(p['ep_ln']['g']), r2(p['ep_ln']['b']),
          p['etype_emb'], r2(p['ge_ln']['g']), r2(p['ge_ln']['b']),
          p['ge_gvp']['Wh'], p['ge_gvp']['Ws'], r2(p['ge_gvp']['bs']),
          p['ge_gvp']['Wv'],
          bd3(wh0[8:9]), ws0[64:96], ws0[160:177], r2(lp['msg0']['bs']),
          bd3(lp['msg0']['Wv']),
          bd3(lp['msg1']['Wh']), lp['msg1']['Ws'], r2(lp['msg1']['bs']),
          bd3(lp['msg1']['Wv']),
          bd3(lp['msg2']['Wh']), lp['msg2']['Ws'], r2(lp['msg2']['bs']),
          bd3(lp['msg2']['Wv'])]

    EH = EP // 2
    aggs = []
    for h in range(2):
        gs, gd = _edge_gather(ts, td, src1[h * EH:(h + 1) * EH],
                              dst1[h * EH:(h + 1) * EH])
        edh = ed[h * EH:min((h + 1) * EH, E)]
        pay = _messages(gs, gd, edh, w3)
        aggs.append(_scatter(pay, dst1[h * EH:(h + 1) * EH]))

    w5 = [r2(lp['norm0']['g']), r2(lp['norm0']['b']),
          bd3(lp['ff0']['Wh']), lp['ff0']['Ws'], r2(lp['ff0']['bs']),
          bd3(lp['ff0']['Wv']),
          bd3(lp['ff1']['Wh']), lp['ff1']['Ws'], r2(lp['ff1']['bs']),
          bd3(lp['ff1']['Wv']),
          r2(lp['norm1']['g']), r2(lp['norm1']['b']),
          r2(p['final_ln']['g']), r2(p['final_ln']['b']),
          bd3(p['out_gvp']['Wh']), p['out_gvp']['Ws'], r2(p['out_gvp']['bs'])]
    out = _node_post(t, aggs[0], aggs[1], w5)
    return out[:N]
